# baseline (speedup 1.0000x reference)
import dataclasses
import functools

import jax
import jax.numpy as jnp
from jax import lax
from jax.experimental import pallas as pl
from jax.experimental.pallas import tpu as pltpu
from jax.experimental.pallas import tpu_sc as plsc

_B = 8
_K = 32
_V = 100000
_HID = 32
_RK = 6
_TEMP = 10.0
_SENTINEL = 2 ** 30

_NC, _NS = 2, 16
_WPR = 4
_VPAD = 100096
_CHUNK = _VPAD // _WPR
_BUF = 25088


def _prep_body(dist_ref, vals_ref, w1_ref, b1_ref, w2_ref, b2_ref,
               idx_ref, w_ref):
    dist = dist_ref[...]
    vals = vals_ref[...]

    vj = vals[:, :, None]
    vm = vals[:, None, :]
    eq = vj == vm
    jidx = lax.broadcasted_iota(jnp.int32, (_B, _K, _K), 1)
    midx = lax.broadcasted_iota(jnp.int32, (_B, _K, _K), 2)
    dup = jnp.any(eq & (midx < jidx), axis=2)

    fo = ((~dup) & (vals != 0)).astype(jnp.float32)
    lc = jnp.sum(jnp.where(midx <= jidx, fo[:, None, :], 0.0), axis=2)

    net_in = jnp.concatenate([dist, lc], axis=1)
    w1 = w1_ref[...]
    h = jnp.tanh(jnp.dot(net_in, w1.T) + b1_ref[...])
    w2 = w2_ref[...]
    logits = jnp.dot(h, w2.T) + b2_ref[...]
    net_out = jax.nn.softmax(logits, axis=-1)
    ksp = net_out[:, 1:]

    kk = lax.broadcasted_iota(jnp.int32, (_RK, _K), 1)
    rr = lax.broadcasted_iota(jnp.int32, (_RK, _K), 0)
    pw = jnp.left_shift(jnp.int32(1), rr)
    kmask = jnp.where(kk < pw, 1.0, 1000.0)
    d = dist[:, None, :] * kmask[None, :, :]
    kw = jax.nn.softmax(-d / _TEMP, axis=-1)
    w = jnp.sum(ksp[:, :, None] * kw, axis=1)

    wsum = jnp.sum(jnp.where(eq, w[:, None, :], 0.0), axis=2)
    idx_ref[...] = jnp.where(dup, _SENTINEL, vals).reshape(_B * _K)
    w_ref[...] = jnp.where(dup, 0.0, wsum).reshape(_B * _K)


_prep = pl.pallas_call(
    _prep_body,
    out_shape=[
        jax.ShapeDtypeStruct((_B * _K,), jnp.int32),
        jax.ShapeDtypeStruct((_B * _K,), jnp.float32),
    ],
)


_HALF = 12544


def _sc_scatter_body(idx_hbm, w_hbm, out_hbm, buf, idxv, wv,
                     sem_i, sem_w, sem_o1, sem_o2):
    wid = lax.axis_index("s") * _NC + lax.axis_index("c")
    row = wid // _WPR
    lo = (wid % _WPR) * _CHUNK

    cp_i = pltpu.async_copy(idx_hbm.at[pl.ds(row * _K, _K)], idxv, sem_i)
    cp_w = pltpu.async_copy(w_hbm.at[pl.ds(row * _K, _K)], wv, sem_w)

    zero = jnp.zeros((16,), jnp.float32)

    @pl.loop(0, _HALF, step=256)
    def _(c):
        for i in range(16):
            buf[pl.ds(c + 16 * i, 16)] = zero

    cp_i.wait()
    cp_w.wait()

    def _scatter(rlo, rhi):
        for h in (0, 16):
            rel = idxv[pl.ds(h, 16)] - lo
            mask = (rel >= rlo) & (rel < rhi)
            relc = jnp.clip(rel, 0, _CHUNK - 1)
            plsc.addupdate_scatter(buf, [relc], wv[pl.ds(h, 16)], mask=mask)

    _scatter(0, _HALF)
    cp_o1 = pltpu.async_copy(buf.at[pl.ds(0, _HALF)],
                             out_hbm.at[row, 0, pl.ds(lo, _HALF)], sem_o1)

    @pl.loop(_HALF, _BUF, step=256)
    def _(c):
        for i in range(16):
            buf[pl.ds(c + 16 * i, 16)] = zero

    _scatter(_HALF, _CHUNK)
    cp_o2 = pltpu.async_copy(
        buf.at[pl.ds(_HALF, _CHUNK - _HALF)],
        out_hbm.at[row, 0, pl.ds(lo + _HALF, _CHUNK - _HALF)], sem_o2)
    cp_o1.wait()
    cp_o2.wait()


@functools.cache
def _sc_scatter():
    cp = pltpu.CompilerParams(use_tc_tiling_on_sc=False,
                              needs_layout_passes=False)
    return pl.kernel(
        _sc_scatter_body,
        compiler_params=cp,
        out_type=jax.ShapeDtypeStruct((_B, 1, _VPAD), jnp.float32),
        mesh=plsc.VectorSubcoreMesh(core_axis_name="c", subcore_axis_name="s"),
        scratch_types=[
            pltpu.VMEM((_BUF,), jnp.float32),
            pltpu.VMEM((_K,), jnp.int32),
            pltpu.VMEM((_K,), jnp.float32),
            pltpu.SemaphoreType.DMA,
            pltpu.SemaphoreType.DMA,
            pltpu.SemaphoreType.DMA,
            pltpu.SemaphoreType.DMA,
        ],
    )


def _sc_probe_body(w_hbm, out_hbm, wv, sem):
    wid = lax.axis_index("s") * _NC + lax.axis_index("c")

    @pl.when(wid == 0)
    def _():
        pltpu.async_copy(w_hbm, wv, sem).wait()
        pltpu.sync_copy(wv, out_hbm)


@functools.cache
def _sc_probe():
    cp = pltpu.CompilerParams(use_tc_tiling_on_sc=False,
                              needs_layout_passes=False)
    return pl.kernel(
        _sc_probe_body,
        compiler_params=cp,
        out_type=jax.ShapeDtypeStruct((_B * _K,), jnp.float32),
        mesh=plsc.VectorSubcoreMesh(core_axis_name="c", subcore_axis_name="s"),
        scratch_types=[
            pltpu.VMEM((_B * _K,), jnp.float32),
            pltpu.SemaphoreType.DMA,
        ],
    )


def kernel(distances, values, W1, b1, W2, b2):
    dist = distances.reshape(_B, _K)
    vals = values.reshape(_B, _K).astype(jnp.int32)
    idxd, wd = _prep(dist, vals, W1, b1.reshape(1, _HID), W2,
                     b2.reshape(1, _RK + 1))
    wd2 = _sc_probe()(wd)
    padded = _sc_scatter()(idxd, wd2)
    return padded[:, :, :_V]


# speedup vs baseline: 1.1954x; 1.1954x over previous
import dataclasses
import functools

import jax
import jax.numpy as jnp
from jax import lax
from jax.experimental import pallas as pl
from jax.experimental.pallas import tpu as pltpu
from jax.experimental.pallas import tpu_sc as plsc

_B = 8
_K = 32
_V = 100000
_HID = 32
_RK = 6
_TEMP = 10.0
_SENTINEL = 2 ** 30

_NC, _NS = 2, 16
_WPR = 4
_VPAD = 100096
_CHUNK = _VPAD // _WPR
_BUF = 25088


def _prep_body(dist_ref, vals_ref, w1_ref, b1_ref, w2_ref, b2_ref,
               idx_ref, w_ref):
    dist = dist_ref[...]
    vals = vals_ref[...]

    vj = vals[:, :, None]
    vm = vals[:, None, :]
    eq = vj == vm
    jidx = lax.broadcasted_iota(jnp.int32, (_B, _K, _K), 1)
    midx = lax.broadcasted_iota(jnp.int32, (_B, _K, _K), 2)
    dup = jnp.any(eq & (midx < jidx), axis=2)

    fo = ((~dup) & (vals != 0)).astype(jnp.float32)
    lc = jnp.sum(jnp.where(midx <= jidx, fo[:, None, :], 0.0), axis=2)

    net_in = jnp.concatenate([dist, lc], axis=1)
    w1 = w1_ref[...]
    h = jnp.tanh(jnp.dot(net_in, w1.T) + b1_ref[...])
    w2 = w2_ref[...]
    logits = jnp.dot(h, w2.T) + b2_ref[...]
    net_out = jax.nn.softmax(logits, axis=-1)
    ksp = net_out[:, 1:]

    kk = lax.broadcasted_iota(jnp.int32, (_RK, _K), 1)
    rr = lax.broadcasted_iota(jnp.int32, (_RK, _K), 0)
    pw = jnp.left_shift(jnp.int32(1), rr)
    kmask = jnp.where(kk < pw, 1.0, 1000.0)
    d = dist[:, None, :] * kmask[None, :, :]
    kw = jax.nn.softmax(-d / _TEMP, axis=-1)
    w = jnp.sum(ksp[:, :, None] * kw, axis=1)

    wsum = jnp.sum(jnp.where(eq, w[:, None, :], 0.0), axis=2)
    idx_ref[...] = jnp.where(dup, _SENTINEL, vals).reshape(_B * _K)
    w_ref[...] = jnp.where(dup, 0.0, wsum).reshape(_B * _K)


_prep = pl.pallas_call(
    _prep_body,
    out_shape=[
        jax.ShapeDtypeStruct((_B * _K,), jnp.int32),
        jax.ShapeDtypeStruct((_B * _K,), jnp.float32),
    ],
)


_HALF = 12544


def _sc_scatter_body(idx_hbm, w_hbm, out_hbm, buf, idxv, wv,
                     sem_i, sem_w, sem_o1, sem_o2):
    wid = lax.axis_index("s") * _NC + lax.axis_index("c")
    row = wid // _WPR
    lo = (wid % _WPR) * _CHUNK

    cp_i = pltpu.async_copy(idx_hbm.at[pl.ds(row * _K, _K)], idxv, sem_i)
    cp_w = pltpu.async_copy(w_hbm.at[pl.ds(row * _K, _K)], wv, sem_w)

    zero = jnp.zeros((16,), jnp.float32)

    @pl.loop(0, _HALF, step=256)
    def _(c):
        for i in range(16):
            buf[pl.ds(c + 16 * i, 16)] = zero

    cp_i.wait()
    cp_w.wait()

    def _scatter(rlo, rhi):
        for h in (0, 16):
            rel = idxv[pl.ds(h, 16)] - lo
            mask = (rel >= rlo) & (rel < rhi)
            relc = jnp.clip(rel, 0, _CHUNK - 1)
            plsc.addupdate_scatter(buf, [relc], wv[pl.ds(h, 16)], mask=mask)

    _scatter(0, _HALF)
    cp_o1 = pltpu.async_copy(buf.at[pl.ds(0, _HALF)],
                             out_hbm.at[row, 0, pl.ds(lo, _HALF)], sem_o1)

    @pl.loop(_HALF, _BUF, step=256)
    def _(c):
        for i in range(16):
            buf[pl.ds(c + 16 * i, 16)] = zero

    _scatter(_HALF, _CHUNK)
    cp_o2 = pltpu.async_copy(
        buf.at[pl.ds(_HALF, _CHUNK - _HALF)],
        out_hbm.at[row, 0, pl.ds(lo + _HALF, _CHUNK - _HALF)], sem_o2)
    cp_o1.wait()
    cp_o2.wait()


@functools.cache
def _sc_scatter():
    cp = pltpu.CompilerParams(use_tc_tiling_on_sc=False,
                              needs_layout_passes=False)
    return pl.kernel(
        _sc_scatter_body,
        compiler_params=cp,
        out_type=jax.ShapeDtypeStruct((_B, 1, _VPAD), jnp.float32),
        mesh=plsc.VectorSubcoreMesh(core_axis_name="c", subcore_axis_name="s"),
        scratch_types=[
            pltpu.VMEM((_BUF,), jnp.float32),
            pltpu.VMEM((_K,), jnp.int32),
            pltpu.VMEM((_K,), jnp.float32),
            pltpu.SemaphoreType.DMA,
            pltpu.SemaphoreType.DMA,
            pltpu.SemaphoreType.DMA,
            pltpu.SemaphoreType.DMA,
        ],
    )


def kernel(distances, values, W1, b1, W2, b2):
    dist = distances.reshape(_B, _K)
    vals = values.reshape(_B, _K).astype(jnp.int32)
    idxd, wd = _prep(dist, vals, W1, b1.reshape(1, _HID), W2,
                     b2.reshape(1, _RK + 1))
    padded = _sc_scatter()(idxd, wd)
    return padded[:, :, :_V]


# speedup vs baseline: 1.2175x; 1.0185x over previous
import functools

import jax
import jax.numpy as jnp
from jax import lax
from jax.experimental import pallas as pl
from jax.experimental.pallas import tpu as pltpu
from jax.experimental.pallas import tpu_sc as plsc

_B = 8
_K = 32
_V = 100000
_HID = 32
_RK = 6
_TEMP = 10.0
_SENTINEL = 2 ** 30

_NC = 2
_WPR = 4
_VPAD = 100096
_CHUNK = _VPAD // _WPR
_BUF = 25088


def _prep_body(dist_ref, vals_ref, w1_ref, b1_ref, w2_ref, b2_ref,
               idx_ref, w_ref):
    dist = dist_ref[...]
    vals = vals_ref[...]

    vj = vals[:, :, None]
    vm = vals[:, None, :]
    eq = vj == vm
    jidx = lax.broadcasted_iota(jnp.int32, (_B, _K, _K), 1)
    midx = lax.broadcasted_iota(jnp.int32, (_B, _K, _K), 2)
    dup = jnp.any(eq & (midx < jidx), axis=2)

    fo = ((~dup) & (vals != 0)).astype(jnp.float32)
    lc = jnp.sum(jnp.where(midx <= jidx, fo[:, None, :], 0.0), axis=2)

    net_in = jnp.concatenate([dist, lc], axis=1)
    w1 = w1_ref[...]
    h = jnp.tanh(jnp.dot(net_in, w1.T) + b1_ref[...])
    w2 = w2_ref[...]
    logits = jnp.dot(h, w2.T) + b2_ref[...]
    net_out = jax.nn.softmax(logits, axis=-1)
    ksp = net_out[:, 1:]

    kk = lax.broadcasted_iota(jnp.int32, (_RK, _K), 1)
    rr = lax.broadcasted_iota(jnp.int32, (_RK, _K), 0)
    pw = jnp.left_shift(jnp.int32(1), rr)
    kmask = jnp.where(kk < pw, 1.0, 1000.0)
    d = dist[:, None, :] * kmask[None, :, :]
    kw = jax.nn.softmax(-d / _TEMP, axis=-1)
    w = jnp.sum(ksp[:, :, None] * kw, axis=1)

    wsum = jnp.sum(jnp.where(eq, w[:, None, :], 0.0), axis=2)
    idx_ref[...] = jnp.where(dup, _SENTINEL, vals).reshape(_B * _K)
    w_ref[...] = jnp.where(dup, 0.0, wsum).reshape(_B * _K)


_prep = pl.pallas_call(
    _prep_body,
    out_shape=[
        jax.ShapeDtypeStruct((_B * _K,), jnp.int32),
        jax.ShapeDtypeStruct((_B * _K,), jnp.float32),
    ],
)


_HALF = 12544


def _sc_scatter_body(idx_hbm, w_hbm, out_hbm, buf, idxv, wv,
                     sem_i, sem_w, sem_o1, sem_o2):
    wid = lax.axis_index("s") * _NC + lax.axis_index("c")
    row = wid // _WPR
    lo = (wid % _WPR) * _CHUNK

    cp_i = pltpu.async_copy(idx_hbm.at[pl.ds(row * _K, _K)], idxv, sem_i)
    cp_w = pltpu.async_copy(w_hbm.at[pl.ds(row * _K, _K)], wv, sem_w)

    zero = jnp.zeros((16,), jnp.float32)

    @pl.loop(0, _HALF, step=256)
    def _(c):
        for i in range(16):
            buf[pl.ds(c + 16 * i, 16)] = zero

    cp_i.wait()
    cp_w.wait()

    def _scatter(rlo, rhi):
        for h in (0, 16):
            rel = idxv[pl.ds(h, 16)] - lo
            mask = (rel >= rlo) & (rel < rhi)
            relc = jnp.clip(rel, 0, _CHUNK - 1)
            plsc.addupdate_scatter(buf, [relc], wv[pl.ds(h, 16)], mask=mask)

    _scatter(0, _HALF)
    cp_o1 = pltpu.async_copy(buf.at[pl.ds(0, _HALF)],
                             out_hbm.at[row, 0, pl.ds(lo, _HALF)], sem_o1)

    @pl.loop(_HALF, _BUF, step=256)
    def _(c):
        for i in range(16):
            buf[pl.ds(c + 16 * i, 16)] = zero

    _scatter(_HALF, _CHUNK)
    cp_o2 = pltpu.async_copy(
        buf.at[pl.ds(_HALF, _CHUNK - _HALF)],
        out_hbm.at[row, 0, pl.ds(lo + _HALF, _CHUNK - _HALF)], sem_o2)
    cp_o1.wait()
    cp_o2.wait()


@functools.cache
def _sc_scatter():
    cp = pltpu.CompilerParams(use_tc_tiling_on_sc=False,
                              needs_layout_passes=False)
    return pl.kernel(
        _sc_scatter_body,
        compiler_params=cp,
        out_type=jax.ShapeDtypeStruct((_B, 1, _VPAD), jnp.float32),
        mesh=plsc.VectorSubcoreMesh(core_axis_name="c", subcore_axis_name="s"),
        scratch_types=[
            pltpu.VMEM((_BUF,), jnp.float32),
            pltpu.VMEM((_K,), jnp.int32),
            pltpu.VMEM((_K,), jnp.float32),
            pltpu.SemaphoreType.DMA,
            pltpu.SemaphoreType.DMA,
            pltpu.SemaphoreType.DMA,
            pltpu.SemaphoreType.DMA,
        ],
    )


def kernel(distances, values, W1, b1, W2, b2):
    dist = distances.reshape(_B, _K)
    vals = values.reshape(_B, _K).astype(jnp.int32)
    idxd, wd = _prep(dist, vals, W1, b1.reshape(1, _HID), W2,
                     b2.reshape(1, _RK + 1))
    padded = _sc_scatter()(idxd, wd)
    return padded[:, :, :_V]
